# Initial kernel scaffold; baseline (speedup 1.0000x reference)
import jax
import jax.numpy as jnp
from jax.experimental import pallas as pl


def kernel(x, edge_index, W1l, W1r, b1, W2l, W2r, b2):
    raise NotImplementedError("write your pallas kernel here")


# speedup vs baseline: 3.5483x; 3.5483x over previous
import jax
import jax.numpy as jnp
from jax import lax
from jax.experimental import pallas as pl
from jax.experimental.pallas import tpu as pltpu
from jax.experimental.pallas import tpu_sc as plsc

N = 10000
E = 160000
D_IN = 256
D_HID = 512
D_OUT = 256

NS = 16
L = 16
W = 128
CH = 80
EPT = E // NS
NCH = EPT // CH
RPT = 640
NP = NS * RPT
RPT_LAST = N - (NS - 1) * RPT
SG = 80

_mesh = plsc.VectorSubcoreMesh(core_axis_name="c", subcore_axis_name="s")


def _count_body(dst, zero_w, one_w, c_out, didx_v, ones_v, acc, sem):
  cid = lax.axis_index("c")
  sid = lax.axis_index("s")

  wbase = pl.multiple_of(sid * RPT, 8)
  pltpu.sync_copy(zero_w, ones_v)
  for j in range(RPT // SG):
    a = pl.multiple_of(wbase + j * SG, 8)
    pltpu.sync_copy(ones_v, acc.at[pl.ds(a, SG)])
  pltpu.sync_copy(one_w, ones_v)
  plsc.subcore_barrier()

  def chunk(c, carry):
    base = pl.multiple_of(sid * EPT + c * CH, 8)
    pltpu.sync_copy(dst.at[pl.ds(base, CH)], didx_v)
    pltpu.sync_copy(ones_v, acc.at[didx_v], add=True)
    return carry

  lax.fori_loop(0, NCH, chunk, 0)
  plsc.subcore_barrier()

  obase = pl.multiple_of(cid * N + sid * RPT, 8)

  def flush(blocks):
    for j in range(blocks):
      a = pl.multiple_of(wbase + j * SG, 8)
      o = pl.multiple_of(obase + j * SG, 8)
      pltpu.sync_copy(acc.at[pl.ds(a, SG)], ones_v)
      pltpu.sync_copy(ones_v, c_out.at[pl.ds(o, SG)])

  @pl.when(sid < NS - 1)
  def _():
    flush(RPT // SG)

  @pl.when(sid == NS - 1)
  def _():
    flush(RPT_LAST // SG)


_sc_count = pl.kernel(
    _count_body,
    out_type=[jax.ShapeDtypeStruct((2 * N, W), jnp.float32)],
    mesh=_mesh,
    scratch_types=[
        pltpu.VMEM((CH,), jnp.int32),
        pltpu.VMEM((CH, W), jnp.float32),
        pltpu.VMEM_SHARED((NP, W), jnp.float32),
        pltpu.SemaphoreType.DMA,
    ],
)


def _mean_body(tab, src2, dst, cnt, zero_w, s_out,
               gidx_v, didx_v, rows_v, cbw_v, acc, sem):
  cid = lax.axis_index("c")
  sid = lax.axis_index("s")

  wbase = pl.multiple_of(sid * RPT, 8)
  pltpu.sync_copy(zero_w, rows_v)
  for j in range(RPT // SG):
    a = pl.multiple_of(wbase + j * SG, 8)
    pltpu.sync_copy(rows_v, acc.at[pl.ds(a, SG)])
  plsc.subcore_barrier()

  def chunk(c, carry):
    base = pl.multiple_of(sid * EPT + c * CH, 8)
    gbase = pl.multiple_of(cid * E + sid * EPT + c * CH, 8)
    pltpu.sync_copy(src2.at[pl.ds(gbase, CH)], gidx_v)
    pltpu.sync_copy(dst.at[pl.ds(base, CH)], didx_v)
    pltpu.async_copy(tab.at[gidx_v], rows_v, sem).wait()
    pltpu.sync_copy(rows_v, acc.at[didx_v], add=True)
    return carry

  lax.fori_loop(0, NCH, chunk, 0)
  plsc.subcore_barrier()

  obase = pl.multiple_of(cid * N + sid * RPT, 8)

  def flush(blocks):
    for j in range(blocks):
      a = pl.multiple_of(wbase + j * SG, 8)
      o = pl.multiple_of(obase + j * SG, 8)
      pltpu.sync_copy(acc.at[pl.ds(a, SG)], rows_v)
      pltpu.sync_copy(cnt.at[pl.ds(a, SG)], cbw_v)

      def rowdiv(r, carry):
        iv = 1.0 / jnp.maximum(cbw_v[r, pl.ds(0, L)], 1.0)
        for k in range(W // L):
          rows_v[r, pl.ds(k * L, L)] = rows_v[r, pl.ds(k * L, L)] * iv
        return carry

      lax.fori_loop(0, SG, rowdiv, 0)
      pltpu.sync_copy(rows_v, s_out.at[pl.ds(o, SG)])

  @pl.when(sid < NS - 1)
  def _():
    flush(RPT // SG)

  @pl.when(sid == NS - 1)
  def _():
    flush(RPT_LAST // SG)


_sc_segmean = pl.kernel(
    _mean_body,
    out_type=[jax.ShapeDtypeStruct((2 * N, W), jnp.float32)],
    mesh=_mesh,
    scratch_types=[
        pltpu.VMEM((CH,), jnp.int32),
        pltpu.VMEM((CH,), jnp.int32),
        pltpu.VMEM((CH, W), jnp.float32),
        pltpu.VMEM((CH, W), jnp.float32),
        pltpu.VMEM_SHARED((NP, W), jnp.float32),
        pltpu.SemaphoreType.DMA,
    ],
)

R = 400
NBLK = N // R


def _tc1_body(m1l, m1r, x, w1l, w1r, b1, w2l, w2r, b2, h1o, p2o, r2o):
  mean = jnp.concatenate([m1l[...], m1r[...]], axis=1)
  h1 = jnp.dot(mean, w1l[...], preferred_element_type=jnp.float32)
  h1 += jnp.dot(x[...], w1r[...], preferred_element_type=jnp.float32)
  h1 = jnp.maximum(h1 + b1[...], 0.0)
  h1o[...] = h1
  p2o[...] = jnp.dot(h1, w2l[...], preferred_element_type=jnp.float32)
  r2o[...] = jnp.dot(h1, w2r[...], preferred_element_type=jnp.float32) + b2[...]


_tc1 = pl.pallas_call(
    _tc1_body,
    grid=(NBLK,),
    in_specs=[
        pl.BlockSpec((R, W), lambda i: (i, 0)),
        pl.BlockSpec((R, W), lambda i: (i + NBLK, 0)),
        pl.BlockSpec((R, D_IN), lambda i: (i, 0)),
        pl.BlockSpec((D_IN, D_HID), lambda i: (0, 0)),
        pl.BlockSpec((D_IN, D_HID), lambda i: (0, 0)),
        pl.BlockSpec((1, D_HID), lambda i: (0, 0)),
        pl.BlockSpec((D_HID, D_OUT), lambda i: (0, 0)),
        pl.BlockSpec((D_HID, D_OUT), lambda i: (0, 0)),
        pl.BlockSpec((1, D_OUT), lambda i: (0, 0)),
    ],
    out_specs=[
        pl.BlockSpec((R, D_HID), lambda i: (i, 0)),
        pl.BlockSpec((R, D_OUT), lambda i: (i, 0)),
        pl.BlockSpec((R, D_OUT), lambda i: (i, 0)),
    ],
    out_shape=[
        jax.ShapeDtypeStruct((N, D_HID), jnp.float32),
        jax.ShapeDtypeStruct((N, D_OUT), jnp.float32),
        jax.ShapeDtypeStruct((N, D_OUT), jnp.float32),
    ],
)


def _tc2_body(m2l, m2r, r2, out):
  out[...] = jnp.concatenate([m2l[...], m2r[...]], axis=1) + r2[...]


_tc2 = pl.pallas_call(
    _tc2_body,
    grid=(NBLK,),
    in_specs=[
        pl.BlockSpec((R, W), lambda i: (i, 0)),
        pl.BlockSpec((R, W), lambda i: (i + NBLK, 0)),
        pl.BlockSpec((R, D_OUT), lambda i: (i, 0)),
    ],
    out_specs=pl.BlockSpec((R, D_OUT), lambda i: (i, 0)),
    out_shape=jax.ShapeDtypeStruct((N, D_OUT), jnp.float32),
)


def kernel(x, edge_index, W1l, W1r, b1, W2l, W2r, b2):
  src = edge_index[0].astype(jnp.int32)
  dst = edge_index[1].astype(jnp.int32)
  src2 = jnp.concatenate([src * 2, src * 2 + 1])
  zero_w = jnp.zeros((SG, W), jnp.float32)
  one_w = jnp.ones((CH, W), jnp.float32)

  (cnt2,) = _sc_count(dst, zero_w, one_w)
  cnt = cnt2[:N]
  (m1,) = _sc_segmean(x.reshape(2 * N, W), src2, dst, cnt, zero_w)
  h1, p2, r2 = _tc1(m1, m1, x, W1l, W1r, b1.reshape(1, D_HID),
                    W2l, W2r, b2.reshape(1, D_OUT))
  (m2,) = _sc_segmean(p2.reshape(2 * N, W), src2, dst, cnt, zero_w)
  out = _tc2(m2, m2, r2)
  return (h1, out)


# speedup vs baseline: 5.0019x; 1.4097x over previous
import jax
import jax.numpy as jnp
from jax import lax
from jax.experimental import pallas as pl
from jax.experimental.pallas import tpu as pltpu
from jax.experimental.pallas import tpu_sc as plsc

N = 10000
E = 160000
D_IN = 256
D_HID = 512
D_OUT = 256

NS = 16
L = 16
W = 128
CH = 80
EPT = E // NS
NCH = EPT // CH
RPT = 640
NP = NS * RPT
RPT_LAST = N - (NS - 1) * RPT
SG = 80

_mesh = plsc.VectorSubcoreMesh(core_axis_name="c", subcore_axis_name="s")


def _count_body(dst, zero_w, one_w, c_out, didx_v, ones_v, acc, sem):
  cid = lax.axis_index("c")
  sid = lax.axis_index("s")

  wbase = pl.multiple_of(sid * RPT, 8)
  pltpu.sync_copy(zero_w, ones_v)
  for j in range(RPT // SG):
    a = pl.multiple_of(wbase + j * SG, 8)
    pltpu.sync_copy(ones_v, acc.at[pl.ds(a, SG)])
  pltpu.sync_copy(one_w, ones_v)
  plsc.subcore_barrier()

  def chunk(c, carry):
    base = pl.multiple_of(sid * EPT + c * CH, 8)
    pltpu.sync_copy(dst.at[pl.ds(base, CH)], didx_v)
    pltpu.sync_copy(ones_v, acc.at[didx_v], add=True)
    return carry

  lax.fori_loop(0, NCH, chunk, 0)
  plsc.subcore_barrier()

  obase = pl.multiple_of(cid * N + sid * RPT, 8)

  def flush(blocks):
    for j in range(blocks):
      a = pl.multiple_of(wbase + j * SG, 8)
      o = pl.multiple_of(obase + j * SG, 8)
      pltpu.sync_copy(acc.at[pl.ds(a, SG)], ones_v)
      pltpu.sync_copy(ones_v, c_out.at[pl.ds(o, SG)])

  @pl.when(sid < NS - 1)
  def _():
    flush(RPT // SG)

  @pl.when(sid == NS - 1)
  def _():
    flush(RPT_LAST // SG)


_sc_count = pl.kernel(
    _count_body,
    out_type=[jax.ShapeDtypeStruct((2 * N, W), jnp.float32)],
    mesh=_mesh,
    scratch_types=[
        pltpu.VMEM((CH,), jnp.int32),
        pltpu.VMEM((CH, W), jnp.float32),
        pltpu.VMEM_SHARED((NP, W), jnp.float32),
        pltpu.SemaphoreType.DMA,
    ],
)


def _mean_body(tab, src2, dst, cnt, zero_w, s_out,
               gidx0, didx0, rows0, gidx1, didx1, rows1, cbw_v, acc,
               sem0, sem1):
  cid = lax.axis_index("c")
  sid = lax.axis_index("s")
  gidx = (gidx0, gidx1)
  didx = (didx0, didx1)
  rows = (rows0, rows1)
  sems = (sem0, sem1)

  wbase = pl.multiple_of(sid * RPT, 8)
  pltpu.sync_copy(zero_w, rows0)
  for j in range(RPT // SG):
    a = pl.multiple_of(wbase + j * SG, 8)
    pltpu.sync_copy(rows0, acc.at[pl.ds(a, SG)])
  plsc.subcore_barrier()

  def issue(c, b):
    base = pl.multiple_of(sid * EPT + c * CH, 8)
    gbase = pl.multiple_of(cid * E + sid * EPT + c * CH, 8)
    pltpu.sync_copy(src2.at[pl.ds(gbase, CH)], gidx[b])
    pltpu.sync_copy(dst.at[pl.ds(base, CH)], didx[b])
    pltpu.async_copy(tab.at[gidx[b]], rows[b], sems[b])

  def drain_scatter(b):
    pltpu.make_async_copy(tab.at[gidx[b]], rows[b], sems[b]).wait()
    pltpu.sync_copy(rows[b], acc.at[didx[b]], add=True)

  issue(0, 0)

  def pair(g, carry):
    issue(2 * g + 1, 1)
    drain_scatter(0)
    issue(2 * g + 2, 0)
    drain_scatter(1)
    return carry

  lax.fori_loop(0, (NCH - 1) // 2, pair, 0)
  drain_scatter(0)
  plsc.subcore_barrier()

  obase = pl.multiple_of(cid * N + sid * RPT, 8)

  def flush(blocks):
    for j in range(blocks):
      a = pl.multiple_of(wbase + j * SG, 8)
      o = pl.multiple_of(obase + j * SG, 8)
      pltpu.sync_copy(acc.at[pl.ds(a, SG)], rows0)
      pltpu.sync_copy(cnt.at[pl.ds(a, SG)], cbw_v)

      def rowdiv(r, carry):
        iv = 1.0 / jnp.maximum(cbw_v[r, pl.ds(0, L)], 1.0)
        for k in range(W // L):
          rows0[r, pl.ds(k * L, L)] = rows0[r, pl.ds(k * L, L)] * iv
        return carry

      lax.fori_loop(0, SG, rowdiv, 0)
      pltpu.sync_copy(rows0, s_out.at[pl.ds(o, SG)])

  @pl.when(sid < NS - 1)
  def _():
    flush(RPT // SG)

  @pl.when(sid == NS - 1)
  def _():
    flush(RPT_LAST // SG)


_sc_segmean = pl.kernel(
    _mean_body,
    out_type=[jax.ShapeDtypeStruct((2 * N, W), jnp.float32)],
    mesh=_mesh,
    scratch_types=[
        pltpu.VMEM((CH,), jnp.int32),
        pltpu.VMEM((CH,), jnp.int32),
        pltpu.VMEM((CH, W), jnp.float32),
        pltpu.VMEM((CH,), jnp.int32),
        pltpu.VMEM((CH,), jnp.int32),
        pltpu.VMEM((CH, W), jnp.float32),
        pltpu.VMEM((CH, W), jnp.float32),
        pltpu.VMEM_SHARED((NP, W), jnp.float32),
        pltpu.SemaphoreType.DMA,
        pltpu.SemaphoreType.DMA,
    ],
)

R = 400
NBLK = N // R


def _tc1_body(m1l, m1r, x, w1l, w1r, b1, w2l, w2r, b2, h1o, p2o, r2o):
  mean = jnp.concatenate([m1l[...], m1r[...]], axis=1)
  h1 = jnp.dot(mean, w1l[...], preferred_element_type=jnp.float32)
  h1 += jnp.dot(x[...], w1r[...], preferred_element_type=jnp.float32)
  h1 = jnp.maximum(h1 + b1[...], 0.0)
  h1o[...] = h1
  p2o[...] = jnp.dot(h1, w2l[...], preferred_element_type=jnp.float32)
  r2o[...] = jnp.dot(h1, w2r[...], preferred_element_type=jnp.float32) + b2[...]


_tc1 = pl.pallas_call(
    _tc1_body,
    grid=(NBLK,),
    in_specs=[
        pl.BlockSpec((R, W), lambda i: (i, 0)),
        pl.BlockSpec((R, W), lambda i: (i + NBLK, 0)),
        pl.BlockSpec((R, D_IN), lambda i: (i, 0)),
        pl.BlockSpec((D_IN, D_HID), lambda i: (0, 0)),
        pl.BlockSpec((D_IN, D_HID), lambda i: (0, 0)),
        pl.BlockSpec((1, D_HID), lambda i: (0, 0)),
        pl.BlockSpec((D_HID, D_OUT), lambda i: (0, 0)),
        pl.BlockSpec((D_HID, D_OUT), lambda i: (0, 0)),
        pl.BlockSpec((1, D_OUT), lambda i: (0, 0)),
    ],
    out_specs=[
        pl.BlockSpec((R, D_HID), lambda i: (i, 0)),
        pl.BlockSpec((R, D_OUT), lambda i: (i, 0)),
        pl.BlockSpec((R, D_OUT), lambda i: (i, 0)),
    ],
    out_shape=[
        jax.ShapeDtypeStruct((N, D_HID), jnp.float32),
        jax.ShapeDtypeStruct((N, D_OUT), jnp.float32),
        jax.ShapeDtypeStruct((N, D_OUT), jnp.float32),
    ],
)


def _tc2_body(m2l, m2r, r2, out):
  out[...] = jnp.concatenate([m2l[...], m2r[...]], axis=1) + r2[...]


_tc2 = pl.pallas_call(
    _tc2_body,
    grid=(NBLK,),
    in_specs=[
        pl.BlockSpec((R, W), lambda i: (i, 0)),
        pl.BlockSpec((R, W), lambda i: (i + NBLK, 0)),
        pl.BlockSpec((R, D_OUT), lambda i: (i, 0)),
    ],
    out_specs=pl.BlockSpec((R, D_OUT), lambda i: (i, 0)),
    out_shape=jax.ShapeDtypeStruct((N, D_OUT), jnp.float32),
)


def kernel(x, edge_index, W1l, W1r, b1, W2l, W2r, b2):
  src = edge_index[0].astype(jnp.int32)
  dst = edge_index[1].astype(jnp.int32)
  src2 = jnp.concatenate([src * 2, src * 2 + 1])
  zero_w = jnp.zeros((SG, W), jnp.float32)
  one_w = jnp.ones((CH, W), jnp.float32)

  (cnt2,) = _sc_count(dst, zero_w, one_w)
  cnt = cnt2[:N]
  (m1,) = _sc_segmean(x.reshape(2 * N, W), src2, dst, cnt, zero_w)
  h1, p2, r2 = _tc1(m1, m1, x, W1l, W1r, b1.reshape(1, D_HID),
                    W2l, W2r, b2.reshape(1, D_OUT))
  (m2,) = _sc_segmean(p2.reshape(2 * N, W), src2, dst, cnt, zero_w)
  out = _tc2(m2, m2, r2)
  return (h1, out)


# speedup vs baseline: 6.2011x; 1.2397x over previous
import jax
import jax.numpy as jnp
from jax import lax
from jax.experimental import pallas as pl
from jax.experimental.pallas import tpu as pltpu
from jax.experimental.pallas import tpu_sc as plsc

N = 10000
E = 160000
D_IN = 256
D_HID = 512
D_OUT = 256

NS = 16
L = 16
W = 128
CH = 80
EPT = E // NS
NCH = EPT // CH
RPT = 640
NP = NS * RPT
RPT_LAST = N - (NS - 1) * RPT
SG = 80

_mesh = plsc.VectorSubcoreMesh(core_axis_name="c", subcore_axis_name="s")


CC = 40
NCC = E // (2 * NS) // CC


def _count_body(dst, zero_w, one_w, c_out, didx_v, ones_v, acc, sem):
  cid = lax.axis_index("c")
  sid = lax.axis_index("s")

  wbase = pl.multiple_of(sid * RPT, 8)
  pltpu.sync_copy(zero_w, ones_v)
  for j in range(RPT // SG):
    a = pl.multiple_of(wbase + j * SG, 8)
    pltpu.sync_copy(ones_v, acc.at[pl.ds(a, SG)])
  pltpu.sync_copy(one_w, ones_v)
  plsc.subcore_barrier()

  ones_cc = ones_v.at[pl.ds(0, CC)]

  def chunk(c, carry):
    base = pl.multiple_of((cid * NS + sid) * (E // (2 * NS)) + c * CC, 8)
    pltpu.sync_copy(dst.at[pl.ds(base, CC)], didx_v)
    pltpu.sync_copy(ones_cc, acc.at[didx_v], add=True)
    return carry

  lax.fori_loop(0, NCC, chunk, 0)
  plsc.subcore_barrier()

  obase = pl.multiple_of(cid * N + sid * RPT, 8)

  def flush(blocks):
    for j in range(blocks):
      a = pl.multiple_of(wbase + j * SG, 8)
      o = pl.multiple_of(obase + j * SG, 8)
      pltpu.sync_copy(acc.at[pl.ds(a, SG)], ones_v)
      pltpu.sync_copy(ones_v, c_out.at[pl.ds(o, SG)])

  @pl.when(sid < NS - 1)
  def _():
    flush(RPT // SG)

  @pl.when(sid == NS - 1)
  def _():
    flush(RPT_LAST // SG)


_sc_count = pl.kernel(
    _count_body,
    out_type=[jax.ShapeDtypeStruct((2 * N, W), jnp.float32)],
    mesh=_mesh,
    scratch_types=[
        pltpu.VMEM((CC,), jnp.int32),
        pltpu.VMEM((CH, W), jnp.float32),
        pltpu.VMEM_SHARED((NP, W), jnp.float32),
        pltpu.SemaphoreType.DMA,
    ],
)


def _mean_body(tab, src2, dst, zero_w, s_out,
               gidx0, didx0, rows0, gidx1, didx1, rows1, acc,
               sem0, sem1, semi0, semi1):
  cid = lax.axis_index("c")
  sid = lax.axis_index("s")
  gidx = (gidx0, gidx1)
  didx = (didx0, didx1)
  rows = (rows0, rows1)
  sems = (sem0, sem1)
  semi = (semi0, semi1)

  wbase = pl.multiple_of(sid * RPT, 8)
  pltpu.sync_copy(zero_w, rows0)
  for j in range(RPT // SG):
    a = pl.multiple_of(wbase + j * SG, 8)
    pltpu.sync_copy(rows0, acc.at[pl.ds(a, SG)])
  plsc.subcore_barrier()

  def issue_idx(c, b, asynchronous=True):
    base = pl.multiple_of(sid * EPT + c * CH, 8)
    gbase = pl.multiple_of(cid * E + sid * EPT + c * CH, 8)
    if asynchronous:
      pltpu.async_copy(src2.at[pl.ds(gbase, CH)], gidx[b], semi[b])
      pltpu.async_copy(dst.at[pl.ds(base, CH)], didx[b], semi[b])
    else:
      pltpu.sync_copy(src2.at[pl.ds(gbase, CH)], gidx[b])
      pltpu.sync_copy(dst.at[pl.ds(base, CH)], didx[b])

  def start_gather(c, b, wait_idx=True):
    if wait_idx:
      gbase = pl.multiple_of(cid * E + sid * EPT + c * CH, 8)
      base = pl.multiple_of(sid * EPT + c * CH, 8)
      pltpu.make_async_copy(src2.at[pl.ds(gbase, CH)], gidx[b],
                            semi[b]).wait()
      pltpu.make_async_copy(dst.at[pl.ds(base, CH)], didx[b],
                            semi[b]).wait()
    pltpu.async_copy(tab.at[gidx[b]], rows[b], sems[b])

  def drain_scatter(b):
    pltpu.make_async_copy(tab.at[gidx[b]], rows[b], sems[b]).wait()
    pltpu.sync_copy(rows[b], acc.at[didx[b]], add=True)

  issue_idx(0, 0, asynchronous=False)
  start_gather(0, 0, wait_idx=False)
  issue_idx(1, 1)

  def pair(g, carry):
    c = 2 * g + 1
    start_gather(c, 1)
    drain_scatter(0)
    issue_idx(c + 1, 0)
    start_gather(c + 1, 0)
    drain_scatter(1)

    @pl.when(c + 2 < NCH)
    def _():
      issue_idx(c + 2, 1)

    return carry

  lax.fori_loop(0, (NCH - 1) // 2, pair, 0)
  drain_scatter(0)
  plsc.subcore_barrier()

  obase = pl.multiple_of(cid * N + sid * RPT, 8)

  def flush(blocks):
    for j in range(blocks):
      a = pl.multiple_of(wbase + j * SG, 8)
      o = pl.multiple_of(obase + j * SG, 8)
      b = j % 2
      if j >= 2:
        o2 = pl.multiple_of(obase + (j - 2) * SG, 8)
        pltpu.make_async_copy(rows[b], s_out.at[pl.ds(o2, SG)],
                              semi[b]).wait()
      pltpu.async_copy(acc.at[pl.ds(a, SG)], rows[b], sems[b])
      pltpu.make_async_copy(acc.at[pl.ds(a, SG)], rows[b], sems[b]).wait()
      pltpu.async_copy(rows[b], s_out.at[pl.ds(o, SG)], semi[b])
    for j in range(max(blocks - 2, 0), blocks):
      o = pl.multiple_of(obase + j * SG, 8)
      pltpu.make_async_copy(rows[j % 2], s_out.at[pl.ds(o, SG)],
                            semi[j % 2]).wait()

  @pl.when(sid < NS - 1)
  def _():
    flush(RPT // SG)

  @pl.when(sid == NS - 1)
  def _():
    flush(RPT_LAST // SG)


_sc_segmean = pl.kernel(
    _mean_body,
    out_type=[jax.ShapeDtypeStruct((2 * N, W), jnp.float32)],
    mesh=_mesh,
    scratch_types=[
        pltpu.VMEM((CH,), jnp.int32),
        pltpu.VMEM((CH,), jnp.int32),
        pltpu.VMEM((CH, W), jnp.float32),
        pltpu.VMEM((CH,), jnp.int32),
        pltpu.VMEM((CH,), jnp.int32),
        pltpu.VMEM((CH, W), jnp.float32),
        pltpu.VMEM_SHARED((NP, W), jnp.float32),
        pltpu.SemaphoreType.DMA,
        pltpu.SemaphoreType.DMA,
        pltpu.SemaphoreType.DMA,
        pltpu.SemaphoreType.DMA,
    ],
)

R = 400
NBLK = N // R


def _tc1_body(s1l, s1r, c0, c1, x, w1l, w1r, b1, w2l, w2r, b2,
              h1o, p2o, r2o):
  iv = 1.0 / jnp.maximum(c0[...][:, 0:1] + c1[...][:, 0:1], 1.0)
  mean = jnp.concatenate([s1l[...], s1r[...]], axis=1) * iv
  h1 = jnp.dot(mean, w1l[...], preferred_element_type=jnp.float32)
  h1 += jnp.dot(x[...], w1r[...], preferred_element_type=jnp.float32)
  h1 = jnp.maximum(h1 + b1[...], 0.0)
  h1o[...] = h1
  p2o[...] = jnp.dot(h1, w2l[...], preferred_element_type=jnp.float32)
  r2o[...] = jnp.dot(h1, w2r[...], preferred_element_type=jnp.float32) + b2[...]


_tc1 = pl.pallas_call(
    _tc1_body,
    grid=(NBLK,),
    in_specs=[
        pl.BlockSpec((R, W), lambda i: (i, 0)),
        pl.BlockSpec((R, W), lambda i: (i + NBLK, 0)),
        pl.BlockSpec((R, W), lambda i: (i, 0)),
        pl.BlockSpec((R, W), lambda i: (i + NBLK, 0)),
        pl.BlockSpec((R, D_IN), lambda i: (i, 0)),
        pl.BlockSpec((D_IN, D_HID), lambda i: (0, 0)),
        pl.BlockSpec((D_IN, D_HID), lambda i: (0, 0)),
        pl.BlockSpec((1, D_HID), lambda i: (0, 0)),
        pl.BlockSpec((D_HID, D_OUT), lambda i: (0, 0)),
        pl.BlockSpec((D_HID, D_OUT), lambda i: (0, 0)),
        pl.BlockSpec((1, D_OUT), lambda i: (0, 0)),
    ],
    out_specs=[
        pl.BlockSpec((R, D_HID), lambda i: (i, 0)),
        pl.BlockSpec((R, D_OUT), lambda i: (i, 0)),
        pl.BlockSpec((R, D_OUT), lambda i: (i, 0)),
    ],
    out_shape=[
        jax.ShapeDtypeStruct((N, D_HID), jnp.float32),
        jax.ShapeDtypeStruct((N, D_OUT), jnp.float32),
        jax.ShapeDtypeStruct((N, D_OUT), jnp.float32),
    ],
)


def _tc2_body(s2l, s2r, c0, c1, r2, out):
  iv = 1.0 / jnp.maximum(c0[...][:, 0:1] + c1[...][:, 0:1], 1.0)
  out[...] = jnp.concatenate([s2l[...], s2r[...]], axis=1) * iv + r2[...]


_tc2 = pl.pallas_call(
    _tc2_body,
    grid=(NBLK,),
    in_specs=[
        pl.BlockSpec((R, W), lambda i: (i, 0)),
        pl.BlockSpec((R, W), lambda i: (i + NBLK, 0)),
        pl.BlockSpec((R, W), lambda i: (i, 0)),
        pl.BlockSpec((R, W), lambda i: (i + NBLK, 0)),
        pl.BlockSpec((R, D_OUT), lambda i: (i, 0)),
    ],
    out_specs=pl.BlockSpec((R, D_OUT), lambda i: (i, 0)),
    out_shape=jax.ShapeDtypeStruct((N, D_OUT), jnp.float32),
)


def kernel(x, edge_index, W1l, W1r, b1, W2l, W2r, b2):
  src = edge_index[0].astype(jnp.int32)
  dst = edge_index[1].astype(jnp.int32)
  src2 = jnp.concatenate([src * 2, src * 2 + 1])
  zero_w = jnp.zeros((SG, W), jnp.float32)
  one_w = jnp.ones((CH, W), jnp.float32)

  (cnt2,) = _sc_count(dst, zero_w, one_w)
  (s1,) = _sc_segmean(x.reshape(2 * N, W), src2, dst, zero_w)
  h1, p2, r2 = _tc1(s1, s1, cnt2, cnt2, x, W1l, W1r, b1.reshape(1, D_HID),
                    W2l, W2r, b2.reshape(1, D_OUT))
  (s2,) = _sc_segmean(p2.reshape(2 * N, W), src2, dst, zero_w)
  out = _tc2(s2, s2, cnt2, cnt2, r2)
  return (h1, out)


# speedup vs baseline: 6.5998x; 1.0643x over previous
import jax
import jax.numpy as jnp
from jax import lax
from jax.experimental import pallas as pl
from jax.experimental.pallas import tpu as pltpu
from jax.experimental.pallas import tpu_sc as plsc

N = 10000
E = 160000
D_IN = 256
D_HID = 512
D_OUT = 256

NS = 16
L = 16
W = 128
CH = 80
EPT = E // NS
NCH = EPT // CH
RPT = 640
NP = NS * RPT
RPT_LAST = N - (NS - 1) * RPT
SG = 80

_mesh = plsc.VectorSubcoreMesh(core_axis_name="c", subcore_axis_name="s")


CC = 40
NCC = E // (2 * NS) // CC


def _count_body(dst, zero_w, one_w, c_out, didx0, didx1, ones_v, acc,
                sem0, sem1):
  cid = lax.axis_index("c")
  sid = lax.axis_index("s")

  wbase = pl.multiple_of(sid * RPT, 8)
  pltpu.sync_copy(zero_w, ones_v)
  for j in range(RPT // SG):
    a = pl.multiple_of(wbase + j * SG, 8)
    pltpu.sync_copy(ones_v, acc.at[pl.ds(a, SG)])
  pltpu.sync_copy(one_w, ones_v)
  plsc.subcore_barrier()

  ones_cc = ones_v.at[pl.ds(0, CC)]
  didx = (didx0, didx1)
  sems = (sem0, sem1)
  ebase = (cid * NS + sid) * (E // (2 * NS))

  def load_idx(c, b):
    base = pl.multiple_of(ebase + c * CC, 8)
    pltpu.sync_copy(dst.at[pl.ds(base, CC)], didx[b])

  def start_scatter(b):
    pltpu.async_copy(ones_cc, acc.at[didx[b]], sems[b], add=True)

  def wait_scatter(b):
    pltpu.make_async_copy(ones_cc, acc.at[didx[b]], sems[b]).wait()

  load_idx(0, 0)
  start_scatter(0)

  def cpair(g, carry):
    load_idx(2 * g + 1, 1)
    wait_scatter(0)
    start_scatter(1)
    load_idx(2 * g + 2, 0)
    wait_scatter(1)
    start_scatter(0)
    return carry

  lax.fori_loop(0, (NCC - 1) // 2, cpair, 0)
  wait_scatter(0)
  plsc.subcore_barrier()

  obase = pl.multiple_of(cid * N + sid * RPT, 8)

  def flush(blocks):
    for j in range(blocks):
      a = pl.multiple_of(wbase + j * SG, 8)
      o = pl.multiple_of(obase + j * SG, 8)
      pltpu.sync_copy(acc.at[pl.ds(a, SG)], ones_v)
      pltpu.sync_copy(ones_v, c_out.at[pl.ds(o, SG)])

  @pl.when(sid < NS - 1)
  def _():
    flush(RPT // SG)

  @pl.when(sid == NS - 1)
  def _():
    flush(RPT_LAST // SG)


_sc_count = pl.kernel(
    _count_body,
    out_type=[jax.ShapeDtypeStruct((2 * N, W), jnp.float32)],
    mesh=_mesh,
    scratch_types=[
        pltpu.VMEM((CC,), jnp.int32),
        pltpu.VMEM((CC,), jnp.int32),
        pltpu.VMEM((CH, W), jnp.float32),
        pltpu.VMEM_SHARED((NP, W), jnp.float32),
        pltpu.SemaphoreType.DMA,
        pltpu.SemaphoreType.DMA,
    ],
)


def _mean_body(tab, src2, dst, zero_w, s_out,
               gidx0, didx0, rows0, gidx1, didx1, rows1, acc,
               sem0, sem1, semi0, semi1):
  cid = lax.axis_index("c")
  sid = lax.axis_index("s")
  gidx = (gidx0, gidx1)
  didx = (didx0, didx1)
  rows = (rows0, rows1)
  sems = (sem0, sem1)
  semi = (semi0, semi1)

  wbase = pl.multiple_of(sid * RPT, 8)
  pltpu.sync_copy(zero_w, rows0)
  for j in range(RPT // SG):
    a = pl.multiple_of(wbase + j * SG, 8)
    pltpu.sync_copy(rows0, acc.at[pl.ds(a, SG)])
  plsc.subcore_barrier()

  def issue_idx(c, b, asynchronous=True):
    base = pl.multiple_of(sid * EPT + c * CH, 8)
    gbase = pl.multiple_of(cid * E + sid * EPT + c * CH, 8)
    if asynchronous:
      pltpu.async_copy(src2.at[pl.ds(gbase, CH)], gidx[b], semi[b])
      pltpu.async_copy(dst.at[pl.ds(base, CH)], didx[b], semi[b])
    else:
      pltpu.sync_copy(src2.at[pl.ds(gbase, CH)], gidx[b])
      pltpu.sync_copy(dst.at[pl.ds(base, CH)], didx[b])

  def start_gather(c, b, wait_idx=True):
    if wait_idx:
      gbase = pl.multiple_of(cid * E + sid * EPT + c * CH, 8)
      base = pl.multiple_of(sid * EPT + c * CH, 8)
      pltpu.make_async_copy(src2.at[pl.ds(gbase, CH)], gidx[b],
                            semi[b]).wait()
      pltpu.make_async_copy(dst.at[pl.ds(base, CH)], didx[b],
                            semi[b]).wait()
    pltpu.async_copy(tab.at[gidx[b]], rows[b], sems[b])

  def drain_scatter(b):
    pltpu.make_async_copy(tab.at[gidx[b]], rows[b], sems[b]).wait()
    pltpu.sync_copy(rows[b], acc.at[didx[b]], add=True)

  issue_idx(0, 0, asynchronous=False)
  start_gather(0, 0, wait_idx=False)
  issue_idx(1, 1)

  def pair(g, carry):
    c = 2 * g + 1
    start_gather(c, 1)
    drain_scatter(0)
    issue_idx(c + 1, 0)
    start_gather(c + 1, 0)
    drain_scatter(1)

    @pl.when(c + 2 < NCH)
    def _():
      issue_idx(c + 2, 1)

    return carry

  lax.fori_loop(0, (NCH - 1) // 2, pair, 0)
  drain_scatter(0)
  plsc.subcore_barrier()

  obase = pl.multiple_of(cid * N + sid * RPT, 8)

  def flush(blocks):
    for j in range(blocks):
      a = pl.multiple_of(wbase + j * SG, 8)
      o = pl.multiple_of(obase + j * SG, 8)
      b = j % 2
      if j >= 2:
        o2 = pl.multiple_of(obase + (j - 2) * SG, 8)
        pltpu.make_async_copy(rows[b], s_out.at[pl.ds(o2, SG)],
                              semi[b]).wait()
      pltpu.async_copy(acc.at[pl.ds(a, SG)], rows[b], sems[b])
      pltpu.make_async_copy(acc.at[pl.ds(a, SG)], rows[b], sems[b]).wait()
      pltpu.async_copy(rows[b], s_out.at[pl.ds(o, SG)], semi[b])
    for j in range(max(blocks - 2, 0), blocks):
      o = pl.multiple_of(obase + j * SG, 8)
      pltpu.make_async_copy(rows[j % 2], s_out.at[pl.ds(o, SG)],
                            semi[j % 2]).wait()

  @pl.when(sid < NS - 1)
  def _():
    flush(RPT // SG)

  @pl.when(sid == NS - 1)
  def _():
    flush(RPT_LAST // SG)


_sc_segmean = pl.kernel(
    _mean_body,
    out_type=[jax.ShapeDtypeStruct((2 * N, W), jnp.float32)],
    mesh=_mesh,
    scratch_types=[
        pltpu.VMEM((CH,), jnp.int32),
        pltpu.VMEM((CH,), jnp.int32),
        pltpu.VMEM((CH, W), jnp.float32),
        pltpu.VMEM((CH,), jnp.int32),
        pltpu.VMEM((CH,), jnp.int32),
        pltpu.VMEM((CH, W), jnp.float32),
        pltpu.VMEM_SHARED((NP, W), jnp.float32),
        pltpu.SemaphoreType.DMA,
        pltpu.SemaphoreType.DMA,
        pltpu.SemaphoreType.DMA,
        pltpu.SemaphoreType.DMA,
    ],
)

R = 400
NBLK = N // R


def _tc1_body(s1l, s1r, c0, c1, x, w1l, w1r, b1, w2l, w2r, b2,
              h1o, p2o, r2o):
  iv = 1.0 / jnp.maximum(c0[...][:, 0:1] + c1[...][:, 0:1], 1.0)
  mean = jnp.concatenate([s1l[...], s1r[...]], axis=1) * iv
  h1 = jnp.dot(mean, w1l[...], preferred_element_type=jnp.float32)
  h1 += jnp.dot(x[...], w1r[...], preferred_element_type=jnp.float32)
  h1 = jnp.maximum(h1 + b1[...], 0.0)
  h1o[...] = h1
  p2o[...] = jnp.dot(h1, w2l[...], preferred_element_type=jnp.float32)
  r2o[...] = jnp.dot(h1, w2r[...], preferred_element_type=jnp.float32) + b2[...]


_tc1 = pl.pallas_call(
    _tc1_body,
    grid=(NBLK,),
    in_specs=[
        pl.BlockSpec((R, W), lambda i: (i, 0)),
        pl.BlockSpec((R, W), lambda i: (i + NBLK, 0)),
        pl.BlockSpec((R, W), lambda i: (i, 0)),
        pl.BlockSpec((R, W), lambda i: (i + NBLK, 0)),
        pl.BlockSpec((R, D_IN), lambda i: (i, 0)),
        pl.BlockSpec((D_IN, D_HID), lambda i: (0, 0)),
        pl.BlockSpec((D_IN, D_HID), lambda i: (0, 0)),
        pl.BlockSpec((1, D_HID), lambda i: (0, 0)),
        pl.BlockSpec((D_HID, D_OUT), lambda i: (0, 0)),
        pl.BlockSpec((D_HID, D_OUT), lambda i: (0, 0)),
        pl.BlockSpec((1, D_OUT), lambda i: (0, 0)),
    ],
    out_specs=[
        pl.BlockSpec((R, D_HID), lambda i: (i, 0)),
        pl.BlockSpec((R, D_OUT), lambda i: (i, 0)),
        pl.BlockSpec((R, D_OUT), lambda i: (i, 0)),
    ],
    out_shape=[
        jax.ShapeDtypeStruct((N, D_HID), jnp.float32),
        jax.ShapeDtypeStruct((N, D_OUT), jnp.float32),
        jax.ShapeDtypeStruct((N, D_OUT), jnp.float32),
    ],
)


def _tc2_body(s2l, s2r, c0, c1, r2, out):
  iv = 1.0 / jnp.maximum(c0[...][:, 0:1] + c1[...][:, 0:1], 1.0)
  out[...] = jnp.concatenate([s2l[...], s2r[...]], axis=1) * iv + r2[...]


_tc2 = pl.pallas_call(
    _tc2_body,
    grid=(NBLK,),
    in_specs=[
        pl.BlockSpec((R, W), lambda i: (i, 0)),
        pl.BlockSpec((R, W), lambda i: (i + NBLK, 0)),
        pl.BlockSpec((R, W), lambda i: (i, 0)),
        pl.BlockSpec((R, W), lambda i: (i + NBLK, 0)),
        pl.BlockSpec((R, D_OUT), lambda i: (i, 0)),
    ],
    out_specs=pl.BlockSpec((R, D_OUT), lambda i: (i, 0)),
    out_shape=jax.ShapeDtypeStruct((N, D_OUT), jnp.float32),
)


def kernel(x, edge_index, W1l, W1r, b1, W2l, W2r, b2):
  src = edge_index[0].astype(jnp.int32)
  dst = edge_index[1].astype(jnp.int32)
  src2 = jnp.concatenate([src * 2, src * 2 + 1])
  zero_w = jnp.zeros((SG, W), jnp.float32)
  one_w = jnp.ones((CH, W), jnp.float32)

  (cnt2,) = _sc_count(dst, zero_w, one_w)
  (s1,) = _sc_segmean(x.reshape(2 * N, W), src2, dst, zero_w)
  h1, p2, r2 = _tc1(s1, s1, cnt2, cnt2, x, W1l, W1r, b1.reshape(1, D_HID),
                    W2l, W2r, b2.reshape(1, D_OUT))
  (s2,) = _sc_segmean(p2.reshape(2 * N, W), src2, dst, zero_w)
  out = _tc2(s2, s2, cnt2, cnt2, r2)
  return (h1, out)
